# Initial kernel scaffold; baseline (speedup 1.0000x reference)
import jax
import jax.numpy as jnp
from jax.experimental import pallas as pl


def kernel(x, w):
    raise NotImplementedError("write your pallas kernel here")


# speedup vs baseline: 1.0229x; 1.0229x over previous
import functools

import jax
import jax.numpy as jnp
from jax import lax
from jax.experimental import pallas as pl
from jax.experimental.pallas import tpu as pltpu
from jax.experimental.pallas import tpu_sc as plsc

EMB_DIM = 32
NC, NS = 2, 16
NW = NC * NS
CHUNK = 128


@functools.lru_cache(maxsize=None)
def _build(n_total: int, vocab: int):
    assert n_total % NW == 0
    per_w = n_total // NW
    assert per_w % CHUNK == 0
    n_chunks = per_w // CHUNK

    mesh = plsc.VectorSubcoreMesh(core_axis_name="c", subcore_axis_name="s")

    @functools.partial(
        pl.kernel,
        out_type=jax.ShapeDtypeStruct((n_total, EMB_DIM), jnp.float32),
        mesh=mesh,
        scratch_types=[
            pltpu.VMEM((per_w,), jnp.int32),
            pltpu.VMEM((CHUNK, EMB_DIM), jnp.float32),
            pltpu.SemaphoreType.DMA,
        ],
        compiler_params=pltpu.CompilerParams(use_tc_tiling_on_sc=False),
    )
    def emb_gather(x_hbm, w_hbm, out_hbm, idx_v, buf_v, sem):
        wid = lax.axis_index("s") * NC + lax.axis_index("c")
        base = wid * per_w
        pltpu.sync_copy(x_hbm.at[pl.ds(base, per_w)], idx_v)

        def body(g, carry):
            idx_sl = idx_v.at[pl.ds(g * CHUNK, CHUNK)]
            pltpu.async_copy(w_hbm.at[idx_sl], buf_v, sem).wait()
            pltpu.sync_copy(buf_v, out_hbm.at[pl.ds(base + g * CHUNK, CHUNK)])
            return carry

        lax.fori_loop(0, n_chunks, body, 0)

    return emb_gather


def kernel(x, w):
    xf = x.reshape(-1).astype(jnp.int32)
    out = _build(xf.shape[0], w.shape[0])(xf, w)
    return out.reshape(x.shape + (EMB_DIM,))


# speedup vs baseline: 1.1103x; 1.0855x over previous
import functools

import jax
import jax.numpy as jnp
from jax import lax
from jax.experimental import pallas as pl
from jax.experimental.pallas import tpu as pltpu
from jax.experimental.pallas import tpu_sc as plsc

EMB_DIM = 32
NC, NS = 2, 16
NW = NC * NS
CHUNK = 128
K = 10
SUPER = CHUNK * K


@functools.lru_cache(maxsize=None)
def _build(n_total: int, vocab: int):
    assert n_total % (NW * SUPER) == 0
    per_w = n_total // NW
    n_super = per_w // SUPER
    assert n_super >= 2 and n_super % 2 == 0

    mesh = plsc.VectorSubcoreMesh(core_axis_name="c", subcore_axis_name="s")

    @functools.partial(
        pl.kernel,
        out_type=jax.ShapeDtypeStruct((n_total, EMB_DIM), jnp.float32),
        mesh=mesh,
        scratch_types=[
            pltpu.VMEM((per_w,), jnp.int32),
            pltpu.VMEM((2, SUPER, EMB_DIM), jnp.float32),
            pltpu.SemaphoreType.DMA,
            pltpu.SemaphoreType.DMA,
            pltpu.SemaphoreType.DMA,
        ],
        compiler_params=pltpu.CompilerParams(use_tc_tiling_on_sc=False),
    )
    def emb_gather(x_hbm, w_hbm, out_hbm, idx_v, buf_v, gsem, ssem0, ssem1):
        wid = lax.axis_index("s") * NC + lax.axis_index("c")
        base = wid * per_w
        pltpu.sync_copy(x_hbm.at[pl.ds(base, per_w)], idx_v)
        ssems = (ssem0, ssem1)

        def gather_super(sup, slot):
            descs = []
            for j in range(K):
                off = sup * SUPER + j * CHUNK
                descs.append(pltpu.async_copy(
                    w_hbm.at[idx_v.at[pl.ds(off, CHUNK)]],
                    buf_v.at[slot, pl.ds(j * CHUNK, CHUNK)],
                    gsem))
            for d in descs:
                d.wait()

        def store_super(sup, slot):
            return pltpu.async_copy(
                buf_v.at[slot], out_hbm.at[pl.ds(base + sup * SUPER, SUPER)],
                ssems[slot])

        def wait_store(sup, slot):
            pltpu.make_async_copy(
                buf_v.at[slot], out_hbm.at[pl.ds(base + sup * SUPER, SUPER)],
                ssems[slot]).wait()

        gather_super(0, 0)
        store_super(0, 0)
        gather_super(1, 1)
        store_super(1, 1)
        wait_store(0, 0)

        def outer(t, carry):
            for s in range(2):
                sup = 2 + 2 * t + s
                gather_super(sup, s)
                store_super(sup, s)
                wait_store(sup - 1, 1 - s)
            return carry

        lax.fori_loop(0, (n_super - 2) // 2, outer, 0)
        wait_store(n_super - 1, 1)

    return emb_gather


def kernel(x, w):
    xf = x.reshape(-1).astype(jnp.int32)
    out = _build(xf.shape[0], w.shape[0])(xf, w)
    return out.reshape(x.shape + (EMB_DIM,))


# speedup vs baseline: 1.7895x; 1.6117x over previous
import functools

import jax
import jax.numpy as jnp
from jax import lax
from jax.experimental import pallas as pl
from jax.experimental.pallas import tpu as pltpu
from jax.experimental.pallas import tpu_sc as plsc

EMB_DIM = 32
NC, NS = 2, 16
NW = NC * NS
SUP_ROWS = 16


@functools.lru_cache(maxsize=None)
def _build(n_rows: int, hist: int, vocab: int):
    assert n_rows % (NW * SUP_ROWS) == 0
    rows_w = n_rows // NW
    n_super = rows_w // SUP_ROWS
    assert n_super >= 2 and n_super % 2 == 0

    mesh = plsc.VectorSubcoreMesh(core_axis_name="c", subcore_axis_name="s")

    @functools.partial(
        pl.kernel,
        out_type=jax.ShapeDtypeStruct((n_rows, hist, EMB_DIM), jnp.float32),
        mesh=mesh,
        scratch_types=[
            pltpu.VMEM((rows_w, hist), jnp.int32),
            pltpu.VMEM((2, SUP_ROWS, hist, EMB_DIM), jnp.float32),
            pltpu.SemaphoreType.DMA,
            pltpu.SemaphoreType.DMA,
            pltpu.SemaphoreType.DMA,
        ],
        compiler_params=pltpu.CompilerParams(use_tc_tiling_on_sc=False),
    )
    def emb_gather(x_hbm, w_hbm, out_hbm, idx_v, buf_v, gsem, ssem0, ssem1):
        wid = lax.axis_index("s") * NC + lax.axis_index("c")
        base = wid * rows_w
        pltpu.sync_copy(x_hbm.at[pl.ds(base, rows_w)], idx_v)
        ssems = (ssem0, ssem1)

        def gather_super(sup, slot):
            descs = []
            for j in range(SUP_ROWS):
                descs.append(pltpu.async_copy(
                    w_hbm.at[idx_v.at[sup * SUP_ROWS + j]],
                    buf_v.at[slot, j],
                    gsem))
            for d in descs:
                d.wait()

        def store_super(sup, slot):
            pltpu.async_copy(
                buf_v.at[slot],
                out_hbm.at[pl.ds(base + sup * SUP_ROWS, SUP_ROWS)],
                ssems[slot])

        def wait_store(sup, slot):
            pltpu.make_async_copy(
                buf_v.at[slot],
                out_hbm.at[pl.ds(base + sup * SUP_ROWS, SUP_ROWS)],
                ssems[slot]).wait()

        gather_super(0, 0)
        store_super(0, 0)
        gather_super(1, 1)
        store_super(1, 1)
        wait_store(0, 0)

        def outer(t, carry):
            for s in range(2):
                sup = 2 + 2 * t + s
                gather_super(sup, s)
                store_super(sup, s)
                wait_store(sup - 1, 1 - s)
            return carry

        lax.fori_loop(0, (n_super - 2) // 2, outer, 0)
        wait_store(n_super - 1, 1)

    return emb_gather


def kernel(x, w):
    xi = x.astype(jnp.int32)
    return _build(xi.shape[0], xi.shape[1], w.shape[0])(xi, w)


# speedup vs baseline: 1.8131x; 1.0132x over previous
import functools

import jax
import jax.numpy as jnp
from jax import lax
from jax.experimental import pallas as pl
from jax.experimental.pallas import tpu as pltpu
from jax.experimental.pallas import tpu_sc as plsc

EMB_DIM = 32
NC, NS = 2, 16
NW = NC * NS
SUP_ROWS = 16


@functools.lru_cache(maxsize=None)
def _build(n_rows: int, hist: int, vocab: int):
    assert n_rows % (NW * SUP_ROWS) == 0
    rows_w = n_rows // NW
    n_super = rows_w // SUP_ROWS
    assert n_super >= 2 and n_super % 2 == 0

    mesh = plsc.VectorSubcoreMesh(core_axis_name="c", subcore_axis_name="s")

    @functools.partial(
        pl.kernel,
        out_type=jax.ShapeDtypeStruct((n_rows, hist, EMB_DIM), jnp.float32),
        mesh=mesh,
        scratch_types=[
            pltpu.VMEM((rows_w, hist), jnp.int32),
            pltpu.VMEM((2, SUP_ROWS, hist, EMB_DIM), jnp.float32),
            pltpu.SemaphoreType.DMA,
            pltpu.SemaphoreType.DMA,
            pltpu.SemaphoreType.DMA,
        ],
        compiler_params=pltpu.CompilerParams(use_tc_tiling_on_sc=False),
    )
    def emb_gather(x_hbm, w_hbm, out_hbm, idx_v, buf_v, gsem, ssem0, ssem1):
        wid = lax.axis_index("s") * NC + lax.axis_index("c")
        base = wid * rows_w
        pltpu.sync_copy(x_hbm.at[pl.ds(base, rows_w)], idx_v)
        ssems = (ssem0, ssem1)

        def gather_super(sup, slot):
            descs = []
            for j in range(SUP_ROWS):
                descs.append(pltpu.async_copy(
                    w_hbm.at[idx_v.at[sup * SUP_ROWS + j]],
                    buf_v.at[slot, j],
                    gsem))
            for d in descs:
                d.wait()

        def store_super(sup, slot):
            pltpu.async_copy(
                buf_v.at[slot],
                out_hbm.at[pl.ds(base + sup * SUP_ROWS, SUP_ROWS)],
                ssems[slot])

        def wait_store(sup, slot):
            pltpu.make_async_copy(
                buf_v.at[slot],
                out_hbm.at[pl.ds(base + sup * SUP_ROWS, SUP_ROWS)],
                ssems[slot]).wait()

        gather_super(0, 0)
        store_super(0, 0)
        gather_super(1, 1)
        store_super(1, 1)
        wait_store(0, 0)

        def outer(t, carry):
            for s in range(2):
                sup = 2 + 2 * t + s
                gather_super(sup, s)
                store_super(sup, s)
                wait_store(sup - 1, 1 - s)
            return carry

        lax.fori_loop(0, (n_super - 2) // 2, outer, 0)
        wait_store(n_super - 1, 1)

    return emb_gather


def kernel(x, w):
    wl = jnp.pad(w, ((0, 0), (0, 128 - EMB_DIM))).reshape(-1, EMB_DIM)
    xi = x.astype(jnp.int32) * 4
    return _build(xi.shape[0], xi.shape[1], wl.shape[0])(xi, wl)
